# baseline (speedup 1.0000x reference)
import functools

import jax
import jax.numpy as jnp
from jax import lax
from jax.experimental import pallas as pl
from jax.experimental.pallas import tpu as pltpu
from jax.experimental.pallas import tpu_sc as plsc

B, T, D, K = 16, 576, 64, 8192
N = B * T
NC, NS, L = 2, 16, 16
NW = NC * NS
BPW = N // NW
CHUNK = 96
NCHUNK = BPW // CHUNK
HALF = N // 2

_mesh = plsc.VectorSubcoreMesh(core_axis_name="c", subcore_axis_name="s")


@functools.partial(
    pl.kernel,
    mesh=_mesh,
    out_type=jax.ShapeDtypeStruct((HALF, 2 * D), jnp.float32),
    scratch_types=[
        pltpu.VMEM((BPW,), jnp.int32),
        pltpu.VMEM((BPW, D), jnp.float32),
        pltpu.SemaphoreType.DMA,
    ],
    compiler_params=pltpu.CompilerParams(
        use_tc_tiling_on_sc=False,
        needs_layout_passes=False,
        skip_device_barrier=True,
    ),
)
def _sc_gather(idx_hbm, table_hbm, q2_hbm, idx_v, rows_v, sem):
    wid = lax.axis_index("s") * NC + lax.axis_index("c")
    base = wid * BPW
    pltpu.sync_copy(idx_hbm.at[pl.ds(base, BPW)], idx_v)
    copies = [
        pltpu.async_copy(
            table_hbm.at[idx_v.at[pl.ds(c * CHUNK, CHUNK)]],
            rows_v.at[pl.ds(c * CHUNK, CHUNK)],
            sem,
        )
        for c in range(NCHUNK)
    ]
    for cp in copies:
        cp.wait()
    row0 = base - (base // HALF) * HALF

    @pl.when(base < HALF)
    def _():
        pltpu.sync_copy(rows_v, q2_hbm.at[pl.ds(row0, BPW), pl.ds(0, D)])

    @pl.when(base >= HALF)
    def _():
        pltpu.sync_copy(rows_v, q2_hbm.at[pl.ds(row0, BPW), pl.ds(D, D)])


_HB = B // 2


def _tc_body(q2_ref, latT_ref, qT_ref, loss_ref):
    q2t = jnp.transpose(q2_ref[...], (1, 0))
    qa, qb = q2t[:D, :], q2t[D:, :]
    da = qa - latT_ref[0, 0]
    db = qb - latT_ref[1, 0]
    qT_ref[0, 0] = qa
    qT_ref[1, 0] = qb
    loss_ref[0, 0, 0, :] = jnp.sum(da * da, axis=0) * (1.25 / D)
    loss_ref[1, 0, 0, :] = jnp.sum(db * db, axis=0) * (1.25 / D)


def _tc_loss_unpack(q2, latT4):
    return pl.pallas_call(
        _tc_body,
        grid=(_HB,),
        in_specs=[
            pl.BlockSpec((T, 2 * D), lambda i: (i, 0)),
            pl.BlockSpec((2, 1, D, T), lambda i: (0, i, 0, 0)),
        ],
        out_specs=[
            pl.BlockSpec((2, 1, D, T), lambda i: (0, i, 0, 0)),
            pl.BlockSpec((2, 1, 1, T), lambda i: (0, i, 0, 0)),
        ],
        out_shape=[
            jax.ShapeDtypeStruct((2, _HB, D, T), jnp.float32),
            jax.ShapeDtypeStruct((2, _HB, 1, T), jnp.float32),
        ],
    )(q2, latT4)


def kernel(gold_encoding_inds, latents, epc, codebook):
    idx = gold_encoding_inds.reshape(N)
    latT4 = latents.transpose(0, 2, 1).reshape(2, _HB, D, T)
    q2 = _sc_gather(idx, codebook)
    qT4, loss4 = _tc_loss_unpack(q2, latT4)
    return (
        qT4.reshape(B, D, T).transpose(0, 2, 1),
        loss4.reshape(B, T),
        gold_encoding_inds.T,
    )


# speedup vs baseline: 1.0466x; 1.0466x over previous
import functools

import jax
import jax.numpy as jnp
from jax import lax
from jax.experimental import pallas as pl
from jax.experimental.pallas import tpu as pltpu
from jax.experimental.pallas import tpu_sc as plsc

B, T, D, K = 16, 576, 64, 8192
N = B * T
NC, NS, L = 2, 16, 16
NW = NC * NS
BPW = N // NW
CHUNK = 96
NCHUNK = BPW // CHUNK
HALF = N // 2

_mesh = plsc.VectorSubcoreMesh(core_axis_name="c", subcore_axis_name="s")


@functools.partial(
    pl.kernel,
    mesh=_mesh,
    out_type=jax.ShapeDtypeStruct((HALF, 2 * D), jnp.float32),
    scratch_types=[
        pltpu.VMEM((BPW,), jnp.int32),
        pltpu.VMEM((BPW, D), jnp.float32),
        pltpu.SemaphoreType.DMA,
        pltpu.SemaphoreType.DMA,
    ],
    compiler_params=pltpu.CompilerParams(
        use_tc_tiling_on_sc=False, needs_layout_passes=False
    ),
)
def _sc_gather(idx_hbm, table_hbm, q2_hbm, idx_v, rows_v, gsem, wsem):
    wid = lax.axis_index("s") * NC + lax.axis_index("c")
    base = wid * BPW
    pltpu.sync_copy(idx_hbm.at[pl.ds(base, BPW)], idx_v)
    copies = [
        pltpu.async_copy(
            table_hbm.at[idx_v.at[pl.ds(c * CHUNK, CHUNK)]],
            rows_v.at[pl.ds(c * CHUNK, CHUNK)],
            gsem,
        )
        for c in range(NCHUNK)
    ]
    half = base // HALF
    row0 = base - half * HALF
    col0 = half * D

    wcps = []
    for c in range(NCHUNK):
        copies[c].wait()
        wcps.append(pltpu.async_copy(
            rows_v.at[pl.ds(c * CHUNK, CHUNK)],
            q2_hbm.at[pl.ds(row0 + c * CHUNK, CHUNK), pl.ds(col0, D)],
            wsem,
        ))
    for wcp in wcps:
        wcp.wait()


_HB = B // 2


def _tc_body(q2_ref, latT_ref, qT_ref, loss_ref):
    q2t = jnp.transpose(q2_ref[...], (1, 0))
    i = pl.program_id(0)
    qa, qb = q2t[:D, :], q2t[D:, :]
    da = qa - latT_ref[0, 0]
    db = qb - latT_ref[1, 0]
    qT_ref[0, 0] = qa
    qT_ref[1, 0] = qb
    loss_ref[0, i] = jnp.sum(da * da, axis=0) * (1.25 / D)
    loss_ref[1, i] = jnp.sum(db * db, axis=0) * (1.25 / D)


def _tc_loss_unpack(q2, latT4):
    return pl.pallas_call(
        _tc_body,
        grid=(_HB,),
        in_specs=[
            pl.BlockSpec((T, 2 * D), lambda i: (i, 0)),
            pl.BlockSpec((2, 1, D, T), lambda i: (0, i, 0, 0)),
        ],
        out_specs=[
            pl.BlockSpec((2, 1, D, T), lambda i: (0, i, 0, 0)),
            pl.BlockSpec((2, _HB, T), lambda i: (0, 0, 0)),
        ],
        out_shape=[
            jax.ShapeDtypeStruct((2, _HB, D, T), jnp.float32),
            jax.ShapeDtypeStruct((2, _HB, T), jnp.float32),
        ],
    )(q2, latT4)


def kernel(gold_encoding_inds, latents, epc, codebook):
    idx = gold_encoding_inds.reshape(N)
    latT4 = latents.transpose(0, 2, 1).reshape(2, _HB, D, T)
    q2 = _sc_gather(idx, codebook)
    qT4, loss4 = _tc_loss_unpack(q2, latT4)
    return (
        qT4.reshape(B, D, T).transpose(0, 2, 1),
        loss4.reshape(B, T),
        gold_encoding_inds.T,
    )


# speedup vs baseline: 1.1146x; 1.0649x over previous
import functools

import jax
import jax.numpy as jnp
from jax import lax
from jax.experimental import pallas as pl
from jax.experimental.pallas import tpu as pltpu
from jax.experimental.pallas import tpu_sc as plsc

B, T, D, K = 16, 576, 64, 8192
N = B * T
NC, NS, L = 2, 16, 16
NW = NC * NS
BPW = N // NW
CHUNK = 96
NCHUNK = BPW // CHUNK
HALF = N // 2

_mesh = plsc.VectorSubcoreMesh(core_axis_name="c", subcore_axis_name="s")


@functools.partial(
    pl.kernel,
    mesh=_mesh,
    out_type=jax.ShapeDtypeStruct((HALF, 2 * D), jnp.float32),
    scratch_types=[
        pltpu.VMEM((BPW,), jnp.int32),
        pltpu.VMEM((BPW, D), jnp.float32),
        pltpu.SemaphoreType.DMA,
        pltpu.SemaphoreType.DMA,
    ],
    compiler_params=pltpu.CompilerParams(
        use_tc_tiling_on_sc=False, needs_layout_passes=False
    ),
)
def _sc_gather(idx_hbm, table_hbm, q2_hbm, idx_v, rows_v, gsem, wsem):
    wid = lax.axis_index("s") * NC + lax.axis_index("c")
    base = wid * BPW
    pltpu.sync_copy(idx_hbm.at[pl.ds(base, BPW)], idx_v)
    copies = [
        pltpu.async_copy(
            table_hbm.at[idx_v.at[pl.ds(c * CHUNK, CHUNK)]],
            rows_v.at[pl.ds(c * CHUNK, CHUNK)],
            gsem,
        )
        for c in range(NCHUNK)
    ]
    half = base // HALF
    row0 = base - half * HALF
    col0 = half * D

    wcps = []
    for c in range(NCHUNK):
        copies[c].wait()
        wcps.append(pltpu.async_copy(
            rows_v.at[pl.ds(c * CHUNK, CHUNK)],
            q2_hbm.at[pl.ds(row0 + c * CHUNK, CHUNK), pl.ds(col0, D)],
            wsem,
        ))
    for wcp in wcps:
        wcp.wait()


_HB = B // 2


_BPS = 2


def _tc_body(q2_ref, latT_ref, qT_ref, loss_ref):
    i = pl.program_id(0)
    for k in range(_BPS):
        q2t = jnp.transpose(q2_ref[pl.ds(k * T, T), :], (1, 0))
        qa, qb = q2t[:D, :], q2t[D:, :]
        da = qa - latT_ref[0, k]
        db = qb - latT_ref[1, k]
        qT_ref[0, k] = qa
        qT_ref[1, k] = qb
        loss_ref[0, i * _BPS + k] = jnp.sum(da * da, axis=0) * (1.25 / D)
        loss_ref[1, i * _BPS + k] = jnp.sum(db * db, axis=0) * (1.25 / D)


def _tc_loss_unpack(q2, latT4):
    return pl.pallas_call(
        _tc_body,
        grid=(_HB // _BPS,),
        in_specs=[
            pl.BlockSpec((_BPS * T, 2 * D), lambda i: (i, 0)),
            pl.BlockSpec((2, _BPS, D, T), lambda i: (0, i, 0, 0)),
        ],
        out_specs=[
            pl.BlockSpec((2, _BPS, D, T), lambda i: (0, i, 0, 0)),
            pl.BlockSpec((2, _HB, T), lambda i: (0, 0, 0)),
        ],
        out_shape=[
            jax.ShapeDtypeStruct((2, _HB, D, T), jnp.float32),
            jax.ShapeDtypeStruct((2, _HB, T), jnp.float32),
        ],
    )(q2, latT4)


def kernel(gold_encoding_inds, latents, epc, codebook):
    idx = gold_encoding_inds.reshape(N)
    latT4 = latents.transpose(0, 2, 1).reshape(2, _HB, D, T)
    q2 = _sc_gather(idx, codebook)
    qT4, loss4 = _tc_loss_unpack(q2, latT4)
    return (
        qT4.reshape(B, D, T).transpose(0, 2, 1),
        loss4.reshape(B, T),
        gold_encoding_inds.T,
    )


# speedup vs baseline: 1.1672x; 1.0471x over previous
import functools

import jax
import jax.numpy as jnp
from jax import lax
from jax.experimental import pallas as pl
from jax.experimental.pallas import tpu as pltpu
from jax.experimental.pallas import tpu_sc as plsc

B, T, D, K = 16, 576, 64, 8192
N = B * T
NC, NS, L = 2, 16, 16
NW = NC * NS
BPW = N // NW
CHUNK = 96
NCHUNK = BPW // CHUNK
HALF = N // 2

_mesh = plsc.VectorSubcoreMesh(core_axis_name="c", subcore_axis_name="s")


@functools.partial(
    pl.kernel,
    mesh=_mesh,
    out_type=jax.ShapeDtypeStruct((HALF, 2 * D), jnp.float32),
    scratch_types=[
        pltpu.VMEM((BPW,), jnp.int32),
        pltpu.VMEM((BPW, D), jnp.float32),
        pltpu.SemaphoreType.DMA,
        pltpu.SemaphoreType.DMA,
    ],
    compiler_params=pltpu.CompilerParams(
        use_tc_tiling_on_sc=False, needs_layout_passes=False
    ),
)
def _sc_gather(idx_hbm, table_hbm, q2_hbm, idx_v, rows_v, gsem, wsem):
    wid = lax.axis_index("s") * NC + lax.axis_index("c")
    base = wid * BPW
    pltpu.sync_copy(idx_hbm.at[pl.ds(base, BPW)], idx_v)
    copies = [
        pltpu.async_copy(
            table_hbm.at[idx_v.at[pl.ds(c * CHUNK, CHUNK)]],
            rows_v.at[pl.ds(c * CHUNK, CHUNK)],
            gsem,
        )
        for c in range(NCHUNK)
    ]
    half = base // HALF
    row0 = base - half * HALF
    col0 = half * D

    wcps = []
    for c in range(NCHUNK):
        copies[c].wait()
        wcps.append(pltpu.async_copy(
            rows_v.at[pl.ds(c * CHUNK, CHUNK)],
            q2_hbm.at[pl.ds(row0 + c * CHUNK, CHUNK), pl.ds(col0, D)],
            wsem,
        ))
    for wcp in wcps:
        wcp.wait()


_HB = B // 2


_BPS = 4


def _tc_body(q2_ref, latT_ref, qT_ref, loss_ref):
    i = pl.program_id(0)
    for k in range(_BPS):
        q2t = jnp.transpose(q2_ref[pl.ds(k * T, T), :], (1, 0))
        qa, qb = q2t[:D, :], q2t[D:, :]
        da = qa - latT_ref[0, k]
        db = qb - latT_ref[1, k]
        qT_ref[0, k] = qa
        qT_ref[1, k] = qb
        loss_ref[0, i * _BPS + k] = jnp.sum(da * da, axis=0) * (1.25 / D)
        loss_ref[1, i * _BPS + k] = jnp.sum(db * db, axis=0) * (1.25 / D)


def _tc_loss_unpack(q2, latT4):
    return pl.pallas_call(
        _tc_body,
        grid=(_HB // _BPS,),
        in_specs=[
            pl.BlockSpec((_BPS * T, 2 * D), lambda i: (i, 0)),
            pl.BlockSpec((2, _BPS, D, T), lambda i: (0, i, 0, 0)),
        ],
        out_specs=[
            pl.BlockSpec((2, _BPS, D, T), lambda i: (0, i, 0, 0)),
            pl.BlockSpec((2, _HB, T), lambda i: (0, 0, 0)),
        ],
        out_shape=[
            jax.ShapeDtypeStruct((2, _HB, D, T), jnp.float32),
            jax.ShapeDtypeStruct((2, _HB, T), jnp.float32),
        ],
    )(q2, latT4)


def kernel(gold_encoding_inds, latents, epc, codebook):
    idx = gold_encoding_inds.reshape(N)
    latT4 = latents.transpose(0, 2, 1).reshape(2, _HB, D, T)
    q2 = _sc_gather(idx, codebook)
    qT4, loss4 = _tc_loss_unpack(q2, latT4)
    return (
        qT4.reshape(B, D, T).transpose(0, 2, 1),
        loss4.reshape(B, T),
        gold_encoding_inds.T,
    )
